# baseline (speedup 1.0000x reference)
import functools

import jax
import jax.numpy as jnp
from jax import lax
from jax.experimental import pallas as pl
from jax.experimental.pallas import tpu as pltpu
from jax.experimental.pallas import tpu_sc as plsc

_N_NODES = 10000
_D = 256
_N_EDGES = 160000
_CHUNK = 80
_N_CHUNKS = _N_EDGES // _CHUNK
_NW = 32
_BASE = _N_CHUNKS // _NW
_EXTRA = _N_CHUNKS % _NW
_IDXCAP = 5632
_E_PAD = 160512


def _sigmoid_body(x_ref, o_ref):
    o_ref[...] = jax.nn.sigmoid(x_ref[...])


def _sigmoid_table(x):
    n, d = x.shape
    blk = 2000
    return pl.pallas_call(
        _sigmoid_body,
        grid=(n // blk,),
        in_specs=[pl.BlockSpec((blk, d), lambda i: (i, 0))],
        out_specs=pl.BlockSpec((blk, d), lambda i: (i, 0)),
        out_shape=jax.ShapeDtypeStruct((n, d), x.dtype),
    )(x)


@functools.partial(
    pl.kernel,
    mesh=plsc.VectorSubcoreMesh(core_axis_name="c", subcore_axis_name="s"),
    out_type=jax.ShapeDtypeStruct((_N_EDGES, 2 * _D), jnp.float32),
    scratch_types=[
        pltpu.VMEM((_IDXCAP,), jnp.int32),
        pltpu.VMEM((_IDXCAP,), jnp.int32),
        pltpu.VMEM((_CHUNK, 2 * _D), jnp.float32),
        pltpu.VMEM((_CHUNK, 2 * _D), jnp.float32),
        pltpu.SemaphoreType.DMA,
        pltpu.SemaphoreType.DMA,
        pltpu.SemaphoreType.DMA,
        pltpu.SemaphoreType.DMA,
    ],
)
def _gather_rows(s_hbm, edge_hbm, out_hbm, idxs_v, idxd_v, b0, b1,
                 g0, g1, o0, o1):
    bufs = (b0, b1)
    gsem = (g0, g1)
    osem = (o0, o1)

    wid = lax.axis_index("s") * 2 + lax.axis_index("c")
    start = wid * _BASE + jnp.minimum(wid, _EXTRA)
    has_extra = wid < _EXTRA
    n = _BASE + has_extra.astype(jnp.int32)

    start_al = (start // 8) * 8
    off = start - start_al
    pltpu.async_copy(
        edge_hbm.at[0, pl.ds(start_al * _CHUNK, _IDXCAP)], idxs_v, g0)
    pltpu.async_copy(
        edge_hbm.at[1, pl.ds(start_al * _CHUNK, _IDXCAP)], idxd_v, g1)
    pltpu.make_async_copy(
        edge_hbm.at[0, pl.ds(start_al * _CHUNK, _IDXCAP)], idxs_v, g0).wait()
    pltpu.make_async_copy(
        edge_hbm.at[1, pl.ds(start_al * _CHUNK, _IDXCAP)], idxd_v, g1).wait()

    def start_gather(j, b):
        sl = pl.ds((off + j) * _CHUNK, _CHUNK)
        pltpu.async_copy(
            s_hbm.at[idxs_v.at[sl]], bufs[b].at[:, pl.ds(0, _D)], gsem[b])
        pltpu.async_copy(
            s_hbm.at[idxd_v.at[sl]], bufs[b].at[:, pl.ds(_D, _D)], gsem[b])

    def wait_gather(j, b):
        sl = pl.ds((off + j) * _CHUNK, _CHUNK)
        pltpu.make_async_copy(
            s_hbm.at[idxs_v.at[sl]], bufs[b].at[:, pl.ds(0, _D)],
            gsem[b]).wait()
        pltpu.make_async_copy(
            s_hbm.at[idxd_v.at[sl]], bufs[b].at[:, pl.ds(_D, _D)],
            gsem[b]).wait()

    def start_scatter(j, b):
        pltpu.async_copy(
            bufs[b], out_hbm.at[pl.ds((start + j) * _CHUNK, _CHUNK)], osem[b])

    def wait_scatter(b):
        pltpu.make_async_copy(
            bufs[b], out_hbm.at[pl.ds(0, _CHUNK)], osem[b]).wait()

    for b in range(2):
        start_gather(b, b)

    def pair(t, carry):
        for b in range(2):
            k = 2 * t + b
            wait_gather(k, b)
            start_scatter(k, b)

            @pl.when(k + 2 < n)
            def _():
                wait_scatter(b)
                start_gather(k + 2, b)

        return carry

    lax.fori_loop(0, _BASE // 2, pair, 0)

    @pl.when(has_extra)
    def _():
        wait_gather(_BASE, _BASE % 2)
        start_scatter(_BASE, _BASE % 2)

    for b in range(2):
        wait_scatter(b)


def kernel(x, edge_index):
    s = _sigmoid_table(x)
    ei = jnp.pad(edge_index.astype(jnp.int32),
                 ((0, 0), (0, _E_PAD - _N_EDGES)))
    return _gather_rows(s, ei)
